# baseline (speedup 1.0000x reference)
import jax
import jax.numpy as jnp
from jax.experimental import pallas as pl
from jax.experimental.pallas import tpu as pltpu

QUEUE_SIZE = 262144
DIM = 128
BATCH = 4096
R = 16384
NBLK = QUEUE_SIZE // R
SUB = 4096
NSUB = R // SUB
NSTEP = NBLK * NSUB
KD_R = BATCH // NSTEP


def _body(p_ref, kpad_ref, q_ref, out_ref, kd_ref):
    i = pl.program_id(0)
    j = pl.program_id(1)
    p = jnp.clip(p_ref[0], 0, QUEUE_SIZE - BATCH)
    sub_start = i * R + j * SUB

    overlap = jnp.logical_and(sub_start + SUB > p, sub_start < p + BATCH)

    @pl.when(overlap)
    def _():
        start = jnp.clip(sub_start - p, -SUB, BATCH) + SUB
        kblk = kpad_ref[pl.ds(start, SUB), :]
        rows = sub_start + jax.lax.broadcasted_iota(jnp.int32, (SUB, 1), 0)
        mask = jnp.logical_and(rows >= p, rows < p + BATCH)
        qsub = q_ref[pl.ds(j * SUB, SUB), :]
        out_ref[...] = jnp.where(mask, kblk, qsub).T

    @pl.when(jnp.logical_not(overlap))
    def _():
        out_ref[...] = q_ref[pl.ds(j * SUB, SUB), :].T

    step = i * NSUB + j
    kd_ref[...] = kpad_ref[pl.ds(SUB + step * KD_R, KD_R), :]


@jax.jit
def _fused(kpad, queue, ptr):
    grid_spec = pltpu.PrefetchScalarGridSpec(
        num_scalar_prefetch=1,
        grid=(NBLK, NSUB),
        in_specs=[
            pl.BlockSpec((BATCH + 2 * SUB, DIM), lambda i, j, p: (0, 0)),
            pl.BlockSpec((R, DIM), lambda i, j, p: (i, 0)),
        ],
        out_specs=[
            pl.BlockSpec((DIM, SUB), lambda i, j, p: (0, i * NSUB + j)),
            pl.BlockSpec((KD_R, DIM), lambda i, j, p: (i * NSUB + j, 0)),
        ],
    )
    return pl.pallas_call(
        _body,
        grid_spec=grid_spec,
        out_shape=[
            jax.ShapeDtypeStruct((DIM, QUEUE_SIZE), jnp.float32),
            jax.ShapeDtypeStruct((BATCH, DIM), jnp.float32),
        ],
    )(ptr, kpad, queue)


def kernel(k, queue, queue_ptr):
    k = jax.lax.stop_gradient(k)
    kpad = jnp.concatenate(
        [
            jnp.zeros((SUB, DIM), jnp.float32),
            k,
            jnp.zeros((SUB, DIM), jnp.float32),
        ]
    )
    ptr = jnp.atleast_1d(jnp.asarray(queue_ptr, jnp.int32))
    queue_t, kd = _fused(kpad, queue, ptr)
    return (kd, queue_t)


# speedup vs baseline: 1.3784x; 1.3784x over previous
import jax
import jax.numpy as jnp
from jax.experimental import pallas as pl
from jax.experimental.pallas import tpu as pltpu

QUEUE_SIZE = 262144
DIM = 128
BATCH = 4096
R = 16384
NBLK = QUEUE_SIZE // R
KD_R = BATCH // NBLK
NSUB = R // BATCH


def _body(p_ref, kpad_ref, q_ref, out_ref, kd_ref):
    i = pl.program_id(0)
    p = jnp.clip(p_ref[0], 0, QUEUE_SIZE - BATCH)
    row_start = i * R

    overlap = jnp.logical_and(row_start + R > p, row_start < p + BATCH)

    @pl.when(overlap)
    def _():
        for j in range(NSUB):
            sub_start = row_start + j * BATCH
            start = jnp.clip(sub_start - p, -BATCH, BATCH) + BATCH
            kblk = kpad_ref[pl.ds(start, BATCH), :]
            rows = sub_start + jax.lax.broadcasted_iota(
                jnp.int32, (BATCH, 1), 0
            )
            mask = jnp.logical_and(rows >= p, rows < p + BATCH)
            qsub = q_ref[pl.ds(j * BATCH, BATCH), :]
            out_ref[:, pl.ds(j * BATCH, BATCH)] = jnp.where(mask, kblk, qsub).T

    @pl.when(jnp.logical_not(overlap))
    def _():
        d = jax.lax.broadcasted_iota(jnp.int32, (DIM, DIM), 0)
        e = jax.lax.broadcasted_iota(jnp.int32, (DIM, DIM), 1)
        eye = jnp.where(d == e, 1.0, 0.0).astype(jnp.float32)
        out_ref[...] = jax.lax.dot_general(
            eye,
            q_ref[...],
            (((1,), (1,)), ((), ())),
            preferred_element_type=jnp.float32,
        )

    kd_ref[...] = kpad_ref[pl.ds(BATCH + i * KD_R, KD_R), :]


@jax.jit
def _fused(kpad, queue, ptr):
    grid_spec = pltpu.PrefetchScalarGridSpec(
        num_scalar_prefetch=1,
        grid=(NBLK,),
        in_specs=[
            pl.BlockSpec((3 * BATCH, DIM), lambda i, p: (0, 0)),
            pl.BlockSpec((R, DIM), lambda i, p: (i, 0)),
        ],
        out_specs=[
            pl.BlockSpec((DIM, R), lambda i, p: (0, i)),
            pl.BlockSpec((KD_R, DIM), lambda i, p: (i, 0)),
        ],
    )
    return pl.pallas_call(
        _body,
        grid_spec=grid_spec,
        compiler_params=pltpu.CompilerParams(vmem_limit_bytes=128 * 1024 * 1024),
        out_shape=[
            jax.ShapeDtypeStruct((DIM, QUEUE_SIZE), jnp.float32),
            jax.ShapeDtypeStruct((BATCH, DIM), jnp.float32),
        ],
    )(ptr, kpad, queue)


def kernel(k, queue, queue_ptr):
    k = jax.lax.stop_gradient(k)
    kpad = jnp.concatenate(
        [
            jnp.zeros((BATCH, DIM), jnp.float32),
            k,
            jnp.zeros((BATCH, DIM), jnp.float32),
        ]
    )
    ptr = jnp.atleast_1d(jnp.asarray(queue_ptr, jnp.int32))
    queue_t, kd = _fused(kpad, queue, ptr)
    return (kd, queue_t)
